# Initial kernel scaffold; baseline (speedup 1.0000x reference)
import jax
import jax.numpy as jnp
from jax.experimental import pallas as pl


def kernel(x, edge_index, edge_attr, batch, Wr, br, Wd, bd, Wm1, bm1, Wih1, Whh1, bih1, bhh1, Wm2, bm2, Wih2, Whh2, bih2, bhh2, Wm3, bm3, Wih3, Whh3, bih3, bhh3, W1, b1, W2, b2):
    raise NotImplementedError("write your pallas kernel here")


# speedup vs baseline: 6.3242x; 6.3242x over previous
import functools

import jax
import jax.numpy as jnp
from jax import lax
from jax.experimental import pallas as pl
from jax.experimental.pallas import tpu as pltpu
from jax.experimental.pallas import tpu_sc as plsc

N = 10000
E = 320000
H = 128
ED = 4
G = 5000
GP = 5008

NC = 2
NS = 16
NW = NC * NS

F32 = jnp.float32
HIGH = jax.lax.Precision.HIGHEST


def _dot(a, b):
    return jax.lax.dot(a, b, precision=HIGH, preferred_element_type=F32)


BN = 1000


def _init_body(x_ref, wrT_ref, br_ref, wdT_ref, bd_ref, wm1T_ref, bm1_ref,
               h0_ref, hp1_ref):
    xb = x_ref[...]
    pr = _dot(xb, wrT_ref[...]) + br_ref[...]
    pd = _dot(xb, wdT_ref[...]) + bd_ref[...]
    row = lax.broadcasted_iota(jnp.int32, (BN, H), 0)
    h0 = jnp.where((row % 2) == 0, pr, pd)
    h0_ref[...] = h0
    hp1_ref[...] = _dot(h0, wm1T_ref[...]) + bm1_ref[...]


def _tc_init(x, wrT, br, wdT, bd, wm1T, bm1):
    grid = (N // BN,)
    wspec = pl.BlockSpec((H, H), lambda i: (0, 0))
    bspec = pl.BlockSpec((1, H), lambda i: (0, 0))
    nspec = pl.BlockSpec((BN, H), lambda i: (i, 0))
    return pl.pallas_call(
        _init_body,
        grid=grid,
        in_specs=[nspec, wspec, bspec, wspec, bspec, wspec, bspec],
        out_specs=[nspec, nspec],
        out_shape=[jax.ShapeDtypeStruct((N, H), F32),
                   jax.ShapeDtypeStruct((N, H), F32)],
    )(x, wrT, br, wdT, bd, wm1T, bm1)


BE = 3200


def _eb_body(ea_ref, w1_ref, w2_ref, w3_ref, e1_ref, e2_ref, e3_ref):
    ea = ea_ref[...]
    e1_ref[...] = _dot(ea, w1_ref[...])
    e2_ref[...] = _dot(ea, w2_ref[...])
    e3_ref[...] = _dot(ea, w3_ref[...])


def _tc_eb(ea8, w1, w2, w3):
    grid = (E // BE,)
    espec = pl.BlockSpec((BE, 8), lambda i: (i, 0))
    wspec = pl.BlockSpec((8, H), lambda i: (0, 0))
    ospec = pl.BlockSpec((BE, H), lambda i: (i, 0))
    return pl.pallas_call(
        _eb_body,
        grid=grid,
        in_specs=[espec, wspec, wspec, wspec],
        out_specs=[ospec, ospec, ospec],
        out_shape=[jax.ShapeDtypeStruct((E, H), F32)] * 3,
    )(ea8, w1, w2, w3)


def _gru_math(agg, h, wihT, bih, whhT, bhh):
    gi = _dot(agg, wihT) + bih
    gh = _dot(h, whhT) + bhh
    r = jax.nn.sigmoid(gi[:, :H] + gh[:, :H])
    z = jax.nn.sigmoid(gi[:, H:2 * H] + gh[:, H:2 * H])
    n = jnp.tanh(gi[:, 2 * H:] + r * gh[:, 2 * H:])
    return jnp.maximum((1.0 - z) * n + z * h, 0.0)


def _gru_next_body(a_ref, h_ref, wihT_ref, bih_ref, whhT_ref, bhh_ref,
                   wmnT_ref, bmn_ref, hout_ref, hpout_ref):
    agg = a_ref[0] + a_ref[1]
    hn = _gru_math(agg, h_ref[...], wihT_ref[...], bih_ref[...],
                   whhT_ref[...], bhh_ref[...])
    hout_ref[...] = hn
    hpout_ref[...] = _dot(hn, wmnT_ref[...]) + bmn_ref[...]


def _gru_last_body(a_ref, h_ref, wihT_ref, bih_ref, whhT_ref, bhh_ref,
                   hout_ref):
    agg = a_ref[0] + a_ref[1]
    hout_ref[...] = _gru_math(agg, h_ref[...], wihT_ref[...], bih_ref[...],
                              whhT_ref[...], bhh_ref[...])


def _tc_gru(parts, h, wihT, bih, whhT, bhh, wmnT=None, bmn=None):
    grid = (N // BN,)
    aspec = pl.BlockSpec((2, BN, H), lambda i: (0, i, 0))
    nspec = pl.BlockSpec((BN, H), lambda i: (i, 0))
    wspec = pl.BlockSpec((H, 3 * H), lambda i: (0, 0))
    bspec = pl.BlockSpec((1, 3 * H), lambda i: (0, 0))
    w2spec = pl.BlockSpec((H, H), lambda i: (0, 0))
    b2spec = pl.BlockSpec((1, H), lambda i: (0, 0))
    if wmnT is None:
        return pl.pallas_call(
            _gru_last_body,
            grid=grid,
            in_specs=[aspec, nspec, wspec, bspec, wspec, bspec],
            out_specs=nspec,
            out_shape=jax.ShapeDtypeStruct((N, H), F32),
        )(parts, h, wihT, bih, whhT, bhh)
    return pl.pallas_call(
        _gru_next_body,
        grid=grid,
        in_specs=[aspec, nspec, wspec, bspec, wspec, bspec, w2spec, b2spec],
        out_specs=[nspec, nspec],
        out_shape=[jax.ShapeDtypeStruct((N, H), F32),
                   jax.ShapeDtypeStruct((N, H), F32)],
    )(parts, h, wihT, bih, whhT, bhh, wmnT, bmn)


BG = 1000


def _mlp_body(p_ref, w1T_ref, b1_ref, w2T_ref, b2_ref, out_ref):
    pooled = p_ref[0] + p_ref[1]
    hid = jnp.maximum(_dot(pooled, w1T_ref[...]) + b1_ref[...], 0.0)
    out_ref[...] = _dot(hid, w2T_ref[...]) + b2_ref[...]


def _tc_mlp(parts, w1T, b1, w2T, b2):
    grid = (G // BG,)
    pspec = pl.BlockSpec((2, BG, H), lambda i: (0, i, 0))
    return pl.pallas_call(
        _mlp_body,
        grid=grid,
        in_specs=[pspec,
                  pl.BlockSpec((H, H // 2), lambda i: (0, 0)),
                  pl.BlockSpec((1, H // 2), lambda i: (0, 0)),
                  pl.BlockSpec((H // 2, H), lambda i: (0, 0)),
                  pl.BlockSpec((1, H), lambda i: (0, 0))],
        out_specs=pl.BlockSpec((BG, H), lambda i: (i, 0)),
        out_shape=jax.ShapeDtypeStruct((G, H), F32),
    )(parts, w1T, b1, w2T, b2)


CE = 80
EPW = E // NW
NCHUNK = EPW // CE
NPT = 624
NREM = N - NS * NPT

_sc_mesh = plsc.VectorSubcoreMesh(core_axis_name="c", subcore_axis_name="s")


def _zero_vmem(buf, rows):
    @pl.loop(0, rows)
    def _(e):
        for j in range(H // 16):
            buf[e, pl.ds(16 * j, 16)] = jnp.zeros((16,), F32)


def _copy_rows(src_ref, dst_ref, base, total, chunk):
    off = 0
    while off < total:
        sz = min(chunk, total - off)
        pltpu.sync_copy(src_ref.at[pl.ds(0, sz), :],
                        dst_ref.at[pl.ds(base + off, sz), :])
        off += sz


@functools.partial(
    pl.kernel,
    out_type=jax.ShapeDtypeStruct((NC, N, H), F32),
    mesh=_sc_mesh,
    scratch_types=[
        pltpu.VMEM((CE,), jnp.int32),
        pltpu.VMEM((CE,), jnp.int32),
        pltpu.VMEM((CE, H), F32),
        pltpu.VMEM((CE, H), F32),
        pltpu.VMEM_SHARED((N, H), F32),
    ],
)
def _sc_edge(hp_hbm, eb_hbm, src_hbm, dst_hbm, out_hbm,
             src_v, dst_v, rows_v, eb_v, acc):
    c = lax.axis_index("c")
    s = lax.axis_index("s")
    wid = c * NS + s
    _zero_vmem(rows_v, CE)
    _copy_rows(rows_v, acc, s * NPT, NPT, CE)

    @pl.when(s == 0)
    def _():
        _copy_rows(rows_v, acc, NS * NPT, NREM, CE)

    plsc.subcore_barrier()
    ebase = wid * EPW

    @pl.loop(0, NCHUNK)
    def _(k):
        b = ebase + k * CE
        pltpu.sync_copy(src_hbm.at[pl.ds(b, CE)], src_v)
        pltpu.sync_copy(dst_hbm.at[pl.ds(b, CE)], dst_v)
        pltpu.sync_copy(eb_hbm.at[pl.ds(b, CE), :], eb_v)
        pltpu.sync_copy(hp_hbm.at[src_v], rows_v)

        @pl.loop(0, CE)
        def _(e):
            for j in range(H // 16):
                sl = pl.ds(16 * j, 16)
                rows_v[e, sl] = jnp.maximum(rows_v[e, sl] + eb_v[e, sl], 0.0)

        pltpu.sync_copy(rows_v, acc.at[dst_v], add=True)

    plsc.subcore_barrier()
    pltpu.sync_copy(acc.at[pl.ds(s * NPT, NPT), :],
                    out_hbm.at[c].at[pl.ds(s * NPT, NPT), :])

    @pl.when(s == 0)
    def _():
        pltpu.sync_copy(acc.at[pl.ds(NS * NPT, NREM), :],
                        out_hbm.at[c].at[pl.ds(NS * NPT, NREM), :])


NPOOL_CHUNKS = N // CE
GPT = 312
GREM = GP - NS * GPT


@functools.partial(
    pl.kernel,
    out_type=jax.ShapeDtypeStruct((NC, GP, H), F32),
    mesh=_sc_mesh,
    scratch_types=[
        pltpu.VMEM((CE,), jnp.int32),
        pltpu.VMEM((CE, H), F32),
        pltpu.VMEM_SHARED((GP, H), F32),
    ],
)
def _sc_pool(h_hbm, batch_hbm, out_hbm, idx_v, rows_v, acc):
    c = lax.axis_index("c")
    s = lax.axis_index("s")
    wid = c * NS + s
    _zero_vmem(rows_v, CE)
    _copy_rows(rows_v, acc, s * GPT, GPT, CE)

    @pl.when(s == 0)
    def _():
        _copy_rows(rows_v, acc, NS * GPT, GREM, CE)

    plsc.subcore_barrier()

    @pl.loop(0, (NPOOL_CHUNKS + NW - 1) // NW)
    def _(p):
        kk = wid + p * NW

        @pl.when(kk < NPOOL_CHUNKS)
        def _():
            b = kk * CE
            pltpu.sync_copy(batch_hbm.at[pl.ds(b, CE)], idx_v)
            pltpu.sync_copy(h_hbm.at[pl.ds(b, CE), :], rows_v)
            pltpu.sync_copy(rows_v, acc.at[idx_v], add=True)

    plsc.subcore_barrier()
    pltpu.sync_copy(acc.at[pl.ds(s * GPT, GPT), :],
                    out_hbm.at[c].at[pl.ds(s * GPT, GPT), :])

    @pl.when(s == 0)
    def _():
        pltpu.sync_copy(acc.at[pl.ds(NS * GPT, GREM), :],
                        out_hbm.at[c].at[pl.ds(NS * GPT, GREM), :])


def kernel(x, edge_index, edge_attr, batch, Wr, br, Wd, bd,
           Wm1, bm1, Wih1, Whh1, bih1, bhh1,
           Wm2, bm2, Wih2, Whh2, bih2, bhh2,
           Wm3, bm3, Wih3, Whh3, bih3, bhh3,
           W1, b1, W2, b2):
    src = edge_index[0]
    dst = edge_index[1]

    RD = Wr.shape[1]
    wrT = jnp.zeros((H, H), F32).at[:RD, :].set(Wr.T)
    wdT = Wd.T
    ea8 = jnp.concatenate([edge_attr, jnp.zeros((E, 8 - ED), F32)], axis=1)

    layers = [(Wm1, bm1, Wih1, Whh1, bih1, bhh1),
              (Wm2, bm2, Wih2, Whh2, bih2, bhh2),
              (Wm3, bm3, Wih3, Whh3, bih3, bhh3)]
    wmHT = [Wm[:, :H].T for Wm, *_ in layers]
    wmET = [jnp.zeros((8, H), F32).at[:ED, :].set(Wm[:, H:].T)
            for Wm, *_ in layers]

    eb1, eb2, eb3 = _tc_eb(ea8, wmET[0], wmET[1], wmET[2])
    ebs = [eb1, eb2, eb3]

    h, hp = _tc_init(x, wrT, br.reshape(1, H), wdT, bd.reshape(1, H),
                     wmHT[0], bm1.reshape(1, H))

    for li, (Wm, bm, Wih, Whh, bih, bhh) in enumerate(layers):
        parts = _sc_edge(hp, ebs[li], src, dst)
        if li < 2:
            nWm, nbm = layers[li + 1][0], layers[li + 1][1]
            h, hp = _tc_gru(parts, h, Wih.T, bih.reshape(1, 3 * H),
                            Whh.T, bhh.reshape(1, 3 * H),
                            wmHT[li + 1], nbm.reshape(1, H))
        else:
            h = _tc_gru(parts, h, Wih.T, bih.reshape(1, 3 * H),
                        Whh.T, bhh.reshape(1, 3 * H))

    pool_parts = _sc_pool(h, batch)[:, :G, :]

    w2T = jnp.zeros((H // 2, H), F32).at[:, 0].set(W2[0])
    b2p = jnp.zeros((1, H), F32).at[0, 0].set(b2[0])
    out = _tc_mlp(pool_parts, W1.T, b1.reshape(1, H // 2), w2T, b2p)
    return out[:, :1]
